# baseline (speedup 1.0000x reference)
import functools

import jax
import jax.numpy as jnp
from jax import lax
from jax.experimental import pallas as pl
from jax.experimental.pallas import tpu as pltpu
from jax.experimental.pallas import tpu_sc as plsc

DIM = 64
RSTRIDE = 1008
EPS2 = 1e-24


def _rsqrt(x):
    i = plsc.bitcast(x, jnp.int32)
    i = jnp.int32(0x5F3759DF) - lax.shift_right_arithmetic(i, 1)
    y = plsc.bitcast(i, jnp.float32)
    for _ in range(3):
        y = y * (1.5 - 0.5 * x * y * y)
    return y


@functools.lru_cache(maxsize=None)
def _make_sc_kernel(n_total: int, n_rels: int, chunk: int):
    info = plsc.get_sparse_core_info()
    nw = info.num_cores * info.num_subcores
    nl = info.num_lanes
    per_w = n_total // nw
    nchunk = per_w // chunk

    mesh = plsc.VectorSubcoreMesh(core_axis_name="c", subcore_axis_name="s")

    @functools.partial(
        pl.kernel,
        mesh=mesh,
        out_type=jax.ShapeDtypeStruct((n_total,), jnp.float32),
        compiler_params=pltpu.CompilerParams(
            needs_layout_passes=False, use_tc_tiling_on_sc=False),
        scratch_types=[
            pltpu.VMEM((per_w,), jnp.int32),
            pltpu.VMEM((per_w,), jnp.int32),
            pltpu.VMEM((per_w,), jnp.int32),
            pltpu.VMEM((DIM * RSTRIDE,), jnp.float32),
            pltpu.VMEM((DIM, chunk), jnp.float32),
            pltpu.VMEM((DIM, chunk), jnp.float32),
            pltpu.VMEM((per_w,), jnp.float32),
            pltpu.SemaphoreType.DMA,
        ],
    )
    def sc_kernel(ents_hbm, rels_hbm, hidx_hbm, ridx_hbm, tidx_hbm, out_hbm,
                  idxh, idxr, idxt, rflat, hbuf, tbuf, scores_v, sem):
        wid = lax.axis_index("s") * info.num_cores + lax.axis_index("c")
        pltpu.sync_copy(hidx_hbm.at[wid], idxh)
        pltpu.sync_copy(ridx_hbm.at[wid], idxr)
        pltpu.sync_copy(tidx_hbm.at[wid], idxt)
        for j in range(DIM):
            pltpu.sync_copy(rels_hbm.at[j],
                            rflat.at[pl.ds(j * RSTRIDE, n_rels)])

        def chunk_body(g, carry):
            base = g * chunk
            hsl = idxh.at[pl.ds(base, chunk)]
            tsl = idxt.at[pl.ds(base, chunk)]
            for j in range(DIM):
                pltpu.async_copy(ents_hbm.at[j].at[hsl],
                                 hbuf.at[j], sem)
                pltpu.async_copy(ents_hbm.at[j].at[tsl],
                                 tbuf.at[j], sem)
            pltpu.make_async_copy(
                ents_hbm.at[:, pl.ds(0, chunk)], hbuf, sem).wait()
            pltpu.make_async_copy(
                ents_hbm.at[:, pl.ds(0, chunk)], tbuf, sem).wait()

            def rb_body(rb, carry2):
                base_r = rb * nl
                sl = pl.ds(base_r, nl)
                ridx16 = idxr[pl.ds(base + base_r, nl)]
                zero = jnp.zeros((nl,), jnp.float32)

                def d_body(j, acc):
                    hh, rr, tt, hr, ht, rt = acc
                    hv = hbuf[j, sl]
                    tv = tbuf[j, sl]
                    rv = plsc.load_gather(rflat, [ridx16 + j * RSTRIDE])
                    return (hh + hv * hv, rr + rv * rv, tt + tv * tv,
                            hr + hv * rv, ht + hv * tv, rt + rv * tv)

                hh, rr, tt, hr, ht, rt = lax.fori_loop(
                    0, DIM, d_body, (zero,) * 6, unroll=8)
                ia = _rsqrt(jnp.maximum(hh, EPS2))
                ib = _rsqrt(jnp.maximum(rr, EPS2))
                ic = _rsqrt(jnp.maximum(tt, EPS2))
                s2 = (hh * ia * ia + rr * ib * ib + tt * ic * ic
                      + 2.0 * (hr * (ia * ib) - ht * (ia * ic)
                               - rt * (ib * ic)))
                s2 = jnp.maximum(s2, 0.0)
                score = s2 * _rsqrt(jnp.maximum(s2, 1e-30))
                scores_v[pl.ds(base + base_r, nl)] = score
                return carry2

            lax.fori_loop(0, chunk // nl, rb_body, 0)
            return carry

        lax.fori_loop(0, nchunk, chunk_body, 0)
        pltpu.sync_copy(scores_v, out_hbm.at[pl.ds(wid * per_w, per_w)])

    return sc_kernel, nw


def kernel(heads, rels, tails, sources, heads_bad, rels_bad, tails_bad,
           sources_bad, ents_weight, rels_weight):
    n = heads.shape[0]
    n_total = 2 * n
    chunk = 256
    sck, nw = _make_sc_kernel(n_total, rels_weight.shape[0], chunk)
    per_w = n_total // nw
    all_heads = jnp.concatenate([heads, heads_bad]).reshape(nw, per_w)
    all_rels = jnp.concatenate([rels, rels_bad]).reshape(nw, per_w)
    all_tails = jnp.concatenate([tails, tails_bad]).reshape(nw, per_w)
    scores = sck(ents_weight.T, rels_weight.T,
                 all_heads, all_rels, all_tails)
    scores = scores.reshape(2, n)
    return (scores[0], scores[1])


# speedup vs baseline: 7.6255x; 7.6255x over previous
import functools

import jax
import jax.numpy as jnp
from jax import lax
from jax.experimental import pallas as pl
from jax.experimental.pallas import tpu as pltpu
from jax.experimental.pallas import tpu_sc as plsc

DIM = 64
EPS2 = 1e-24


def _rsqrt(x):
    i = plsc.bitcast(x, jnp.int32)
    i = jnp.int32(0x5F3759DF) - lax.shift_right_arithmetic(i, 1)
    y = plsc.bitcast(i, jnp.float32)
    for _ in range(3):
        y = y * (1.5 - 0.5 * x * y * y)
    return y


def _tree_sum(vs):
    while len(vs) > 1:
        vs = [a + b for a, b in zip(vs[::2], vs[1::2])]
    return vs[0]


@functools.lru_cache(maxsize=None)
def _make_sc_kernel(n_total: int, chunk: int):
    info = plsc.get_sparse_core_info()
    nw = info.num_cores * info.num_subcores
    nl = info.num_lanes
    per_w = n_total // nw
    nchunk = per_w // chunk

    mesh = plsc.VectorSubcoreMesh(core_axis_name="c", subcore_axis_name="s")

    @functools.partial(
        pl.kernel,
        mesh=mesh,
        out_type=jax.ShapeDtypeStruct((n_total,), jnp.float32),
        compiler_params=pltpu.CompilerParams(needs_layout_passes=False),
        scratch_types=[
            pltpu.VMEM((per_w,), jnp.int32),
            pltpu.VMEM((per_w,), jnp.int32),
            pltpu.VMEM((per_w,), jnp.int32),
            pltpu.VMEM((per_w,), jnp.int32),
            pltpu.VMEM((per_w,), jnp.int32),
            pltpu.VMEM((per_w,), jnp.int32),
            pltpu.VMEM((chunk, 2 * DIM), jnp.float32),
            pltpu.VMEM((chunk, 2 * DIM), jnp.float32),
            pltpu.VMEM((chunk, 2 * DIM), jnp.float32),
            pltpu.VMEM((nl * 6 * nl,), jnp.float32),
            pltpu.VMEM((per_w,), jnp.float32),
            pltpu.SemaphoreType.DMA,
        ],
    )
    def sc_kernel(ents_hbm, rels_hbm, hidx_hbm, ridx_hbm, tidx_hbm, out_hbm,
                  pixh, pixr, pixt, parh, parr, part,
                  hbuf, rbuf, tbuf, stage, scores_v, sem):
        wid = lax.axis_index("s") * info.num_cores + lax.axis_index("c")
        pltpu.sync_copy(hidx_hbm.at[wid], pixh)
        pltpu.sync_copy(ridx_hbm.at[wid], pixr)
        pltpu.sync_copy(tidx_hbm.at[wid], pixt)
        lanes = lax.iota(jnp.int32, nl)
        lanes_cols = lanes * (6 * nl)

        def split_body(q, carry):
            qb = q * nl
            for pix, par in ((pixh, parh), (pixr, parr), (pixt, part)):
                v = pix[pl.ds(qb, nl)]
                par[pl.ds(qb, nl)] = (v & 1) * DIM
                pix[pl.ds(qb, nl)] = lax.shift_right_logical(v, 1)
            return carry

        lax.fori_loop(0, per_w // nl, split_body, 0)

        def chunk_body(g, carry):
            base = g * chunk
            ch = pltpu.async_copy(
                ents_hbm.at[pixh.at[pl.ds(base, chunk)]], hbuf, sem)
            cr = pltpu.async_copy(
                rels_hbm.at[pixr.at[pl.ds(base, chunk)]], rbuf, sem)
            ct = pltpu.async_copy(
                ents_hbm.at[pixt.at[pl.ds(base, chunk)]], tbuf, sem)
            ch.wait()
            cr.wait()
            ct.wait()

            def rb_body(rb, carry2):
                base_r = rb * nl
                ph = parh[pl.ds(base + base_r, nl)]
                pr = parr[pl.ds(base + base_r, nl)]
                pt = part[pl.ds(base + base_r, nl)]
                for rm in range(nl):
                    r = base_r + rm
                    oh, orr, ot = ph[rm], pr[rm], pt[rm]
                    h = [hbuf[r, pl.ds(oh + j * nl, nl)]
                         for j in range(DIM // nl)]
                    rv = [rbuf[r, pl.ds(orr + j * nl, nl)]
                          for j in range(DIM // nl)]
                    t = [tbuf[r, pl.ds(ot + j * nl, nl)]
                         for j in range(DIM // nl)]
                    prods = (
                        _tree_sum([x * x for x in h]),
                        _tree_sum([x * x for x in rv]),
                        _tree_sum([x * x for x in t]),
                        _tree_sum([x * y for x, y in zip(h, rv)]),
                        _tree_sum([x * y for x, y in zip(h, t)]),
                        _tree_sum([x * y for x, y in zip(rv, t)]),
                    )
                    for k, v in enumerate(prods):
                        plsc.store_scatter(
                            stage, [lanes_cols + (k * nl + rm)], v)

                tot = [
                    _tree_sum([stage[pl.ds(j * 6 * nl + k * nl, nl)]
                               for j in range(nl)])
                    for k in range(6)
                ]
                hh, rr, tt, hr, ht, rt = tot
                ia = _rsqrt(jnp.maximum(hh, EPS2))
                ib = _rsqrt(jnp.maximum(rr, EPS2))
                ic = _rsqrt(jnp.maximum(tt, EPS2))
                s2 = (hh * ia * ia + rr * ib * ib + tt * ic * ic
                      + 2.0 * (hr * (ia * ib) - ht * (ia * ic)
                               - rt * (ib * ic)))
                s2 = jnp.maximum(s2, 0.0)
                score = s2 * _rsqrt(jnp.maximum(s2, 1e-30))
                scores_v[pl.ds(base + base_r, nl)] = score
                return carry2

            lax.fori_loop(0, chunk // nl, rb_body, 0)
            return carry

        lax.fori_loop(0, nchunk, chunk_body, 0)
        pltpu.sync_copy(scores_v, out_hbm.at[pl.ds(wid * per_w, per_w)])

    return sc_kernel, nw


def kernel(heads, rels, tails, sources, heads_bad, rels_bad, tails_bad,
           sources_bad, ents_weight, rels_weight):
    n = heads.shape[0]
    n_total = 2 * n
    chunk = 128
    sck, nw = _make_sc_kernel(n_total, chunk)
    per_w = n_total // nw
    n_ents = ents_weight.shape[0] - 1
    n_rels = rels_weight.shape[0] - 1
    ents2 = ents_weight[:n_ents].reshape(n_ents // 2, 2 * DIM)
    rels2 = rels_weight[:n_rels].reshape(n_rels // 2, 2 * DIM)
    all_heads = jnp.concatenate([heads, heads_bad]).reshape(nw, per_w)
    all_rels = jnp.concatenate([rels, rels_bad]).reshape(nw, per_w)
    all_tails = jnp.concatenate([tails, tails_bad]).reshape(nw, per_w)
    scores = sck(ents2, rels2, all_heads, all_rels, all_tails)
    scores = scores.reshape(2, n)
    return (scores[0], scores[1])


# speedup vs baseline: 12.0329x; 1.5780x over previous
import functools

import jax
import jax.numpy as jnp
from jax import lax
from jax.experimental import pallas as pl
from jax.experimental.pallas import tpu as pltpu
from jax.experimental.pallas import tpu_sc as plsc

DIM = 64
EPS2 = 1e-24


def _rsqrt(x):
    i = plsc.bitcast(x, jnp.int32)
    i = jnp.int32(0x5F3759DF) - lax.shift_right_arithmetic(i, 1)
    y = plsc.bitcast(i, jnp.float32)
    for _ in range(3):
        y = y * (1.5 - 0.5 * x * y * y)
    return y


def _tree_sum(vs):
    while len(vs) > 1:
        vs = [a + b for a, b in zip(vs[::2], vs[1::2])]
    return vs[0]


@functools.lru_cache(maxsize=None)
def _make_sc_kernel(n_total: int, chunk: int):
    info = plsc.get_sparse_core_info()
    nw = info.num_cores * info.num_subcores
    nl = info.num_lanes
    per_w = n_total // nw
    nchunk = per_w // chunk
    assert nchunk % 2 == 0

    mesh = plsc.VectorSubcoreMesh(core_axis_name="c", subcore_axis_name="s")

    @functools.partial(
        pl.kernel,
        mesh=mesh,
        out_type=jax.ShapeDtypeStruct((n_total,), jnp.float32),
        compiler_params=pltpu.CompilerParams(needs_layout_passes=False),
        scratch_types=[
            pltpu.VMEM((per_w,), jnp.int32),
            pltpu.VMEM((per_w,), jnp.int32),
            pltpu.VMEM((per_w,), jnp.int32),
            pltpu.VMEM((chunk, DIM), jnp.float32),
            pltpu.VMEM((chunk, DIM), jnp.float32),
            pltpu.VMEM((chunk, DIM), jnp.float32),
            pltpu.VMEM((chunk, DIM), jnp.float32),
            pltpu.VMEM((chunk, DIM), jnp.float32),
            pltpu.VMEM((chunk, DIM), jnp.float32),
            pltpu.VMEM((nl * 6 * nl,), jnp.float32),
            pltpu.VMEM((per_w,), jnp.float32),
            pltpu.SemaphoreType.DMA,
            pltpu.SemaphoreType.DMA,
        ],
    )
    def sc_kernel(ents_hbm, rels_hbm, hidx_hbm, ridx_hbm, tidx_hbm, out_hbm,
                  idxh, idxr, idxt, hbufa, rbufa, tbufa, hbufb, rbufb, tbufb,
                  stage, scores_v, sema, semb):
        wid = lax.axis_index("s") * info.num_cores + lax.axis_index("c")
        pltpu.sync_copy(hidx_hbm.at[wid], idxh)
        pltpu.sync_copy(ridx_hbm.at[wid], idxr)
        pltpu.sync_copy(tidx_hbm.at[wid], idxt)
        lanes = lax.iota(jnp.int32, nl)
        lanes_cols = lanes * (6 * nl)

        def fire(g, hb, rb_, tb, s):
            base = g * chunk

            def dma_body(q, c2):
                qb = q * nl
                vh = idxh[pl.ds(base + qb, nl)]
                vr = idxr[pl.ds(base + qb, nl)]
                vt = idxt[pl.ds(base + qb, nl)]
                for rm in range(nl):
                    pltpu.async_copy(
                        ents_hbm.at[pl.ds(vh[rm], 1)],
                        hb.at[pl.ds(qb + rm, 1)], s)
                    pltpu.async_copy(
                        rels_hbm.at[pl.ds(vr[rm], 1)],
                        rb_.at[pl.ds(qb + rm, 1)], s)
                    pltpu.async_copy(
                        ents_hbm.at[pl.ds(vt[rm], 1)],
                        tb.at[pl.ds(qb + rm, 1)], s)
                return c2

            lax.fori_loop(0, chunk // nl, dma_body, 0)

        def drain(hb, rb_, tb, s):
            pltpu.make_async_copy(ents_hbm.at[pl.ds(0, chunk)], hb, s).wait()
            pltpu.make_async_copy(ents_hbm.at[pl.ds(0, chunk)], rb_, s).wait()
            pltpu.make_async_copy(ents_hbm.at[pl.ds(0, chunk)], tb, s).wait()

        def compute(g, hb, rb_, tb):
            base = g * chunk

            def rb_body(rb, carry2):
                base_r = rb * nl
                for rm in range(nl):
                    r = base_r + rm
                    h = [hb[r, pl.ds(j * nl, nl)] for j in range(DIM // nl)]
                    rv = [rb_[r, pl.ds(j * nl, nl)] for j in range(DIM // nl)]
                    t = [tb[r, pl.ds(j * nl, nl)] for j in range(DIM // nl)]
                    prods = (
                        _tree_sum([x * x for x in h]),
                        _tree_sum([x * x for x in rv]),
                        _tree_sum([x * x for x in t]),
                        _tree_sum([x * y for x, y in zip(h, rv)]),
                        _tree_sum([x * y for x, y in zip(h, t)]),
                        _tree_sum([x * y for x, y in zip(rv, t)]),
                    )
                    for k, v in enumerate(prods):
                        plsc.store_scatter(
                            stage, [lanes_cols + (k * nl + rm)], v)

                tot = [
                    _tree_sum([stage[pl.ds(j * 6 * nl + k * nl, nl)]
                               for j in range(nl)])
                    for k in range(6)
                ]
                hh, rr, tt, hr, ht, rt = tot
                ia = _rsqrt(jnp.maximum(hh, EPS2))
                ib = _rsqrt(jnp.maximum(rr, EPS2))
                ic = _rsqrt(jnp.maximum(tt, EPS2))
                s2 = (hh * ia * ia + rr * ib * ib + tt * ic * ic
                      + 2.0 * (hr * (ia * ib) - ht * (ia * ic)
                               - rt * (ib * ic)))
                s2 = jnp.maximum(s2, 0.0)
                score = s2 * _rsqrt(jnp.maximum(s2, 1e-30))
                scores_v[pl.ds(base + base_r, nl)] = score
                return carry2

            lax.fori_loop(0, chunk // nl, rb_body, 0)

        fire(0, hbufa, rbufa, tbufa, sema)

        def pair_body(tpair, carry):
            ge = 2 * tpair
            go = ge + 1
            fire(go, hbufb, rbufb, tbufb, semb)
            drain(hbufa, rbufa, tbufa, sema)
            compute(ge, hbufa, rbufa, tbufa)

            @pl.when(tpair + 1 < nchunk // 2)
            def _():
                fire(go + 1, hbufa, rbufa, tbufa, sema)

            drain(hbufb, rbufb, tbufb, semb)
            compute(go, hbufb, rbufb, tbufb)
            return carry

        lax.fori_loop(0, nchunk // 2, pair_body, 0)
        pltpu.sync_copy(scores_v, out_hbm.at[pl.ds(wid * per_w, per_w)])

    return sc_kernel, nw


def kernel(heads, rels, tails, sources, heads_bad, rels_bad, tails_bad,
           sources_bad, ents_weight, rels_weight):
    n = heads.shape[0]
    n_total = 2 * n
    chunk = 128
    sck, nw = _make_sc_kernel(n_total, chunk)
    per_w = n_total // nw
    all_heads = jnp.concatenate([heads, heads_bad]).reshape(nw, per_w)
    all_rels = jnp.concatenate([rels, rels_bad]).reshape(nw, per_w)
    all_tails = jnp.concatenate([tails, tails_bad]).reshape(nw, per_w)
    scores = sck(ents_weight, rels_weight, all_heads, all_rels, all_tails)
    scores = scores.reshape(2, n)
    return (scores[0], scores[1])
